# baseline (speedup 1.0000x reference)
import jax
import jax.numpy as jnp
from jax import lax
from jax.experimental import pallas as pl
from jax.experimental.pallas import tpu as pltpu

B, H, W = 2, 224, 224
K, D = 19, 1024
P = H * W
KPAD = 32
DT = 32
NJ = D // DT


def _body(src_ref, colors_ref, table_ref, o1_ref, o2_ref, onehot_ref):
    @pl.when(pl.program_id(0) == 0)
    def _build_onehot():
        for b in range(B):
            q = (src_ref[b] * 127.5 + 127.5).astype(jnp.int32)
            match = None
            for c in range(3):
                eq = q[c:c + 1, :] == colors_ref[:, c:c + 1]
                match = eq if match is None else (match & eq)
            kvec = lax.broadcasted_iota(jnp.int32, (K, P), 0)
            cls = jnp.min(jnp.where(match, kvec, KPAD - 1), axis=0,
                          keepdims=True)
            onehot_ref[b] = (
                cls == lax.broadcasted_iota(jnp.int32, (KPAD, P), 0)
            ).astype(jnp.float32)

    j = pl.program_id(0)
    tb = table_ref[pl.ds(j * DT, DT), :]
    for b, out_ref in ((0, o1_ref), (1, o2_ref)):
        out_ref[...] = lax.dot_general(
            tb, onehot_ref[b], (((1,), (0,)), ((), ())),
            preferred_element_type=jnp.float32)


def kernel(src, colors, feats):
    src_flat = src.reshape(B, 3, P)
    colors_i = colors.astype(jnp.int32)
    table = jnp.zeros((D, KPAD), jnp.float32).at[:, :K].set(feats.T)
    o1, o2 = pl.pallas_call(
        _body,
        grid=(NJ,),
        in_specs=[
            pl.BlockSpec((B, 3, P), lambda j: (0, 0, 0)),
            pl.BlockSpec((K, 3), lambda j: (0, 0)),
            pl.BlockSpec((D, KPAD), lambda j: (0, 0)),
        ],
        out_specs=[pl.BlockSpec((DT, P), lambda j: (j, 0)),
                   pl.BlockSpec((DT, P), lambda j: (j, 0))],
        out_shape=[jax.ShapeDtypeStruct((D, P), jnp.float32),
                   jax.ShapeDtypeStruct((D, P), jnp.float32)],
        scratch_shapes=[pltpu.VMEM((B, KPAD, P), jnp.float32)],
        compiler_params=pltpu.CompilerParams(
            dimension_semantics=("arbitrary",)),
    )(src_flat, colors_i, table)
    out = jnp.stack([o1, o2], axis=0)
    return out.reshape(B, D, H, W)
